# baseline (speedup 1.0000x reference)
import functools
import math

import jax
import jax.numpy as jnp
from jax import lax
from jax.experimental import pallas as pl
from jax.experimental.pallas import tpu as pltpu
from jax.experimental.pallas import tpu_sc as plsc

_SCALE = float(math.sqrt(32.0))

_NC = 2
_NS = 16
_NW = _NC * _NS
_L = 16

_MESH = plsc.VectorSubcoreMesh(core_axis_name="c", subcore_axis_name="s")


def _wid():
    return lax.axis_index("s") * _NC + lax.axis_index("c")


def _iota():
    return lax.iota(jnp.int32, _L)


def _splat(x):
    return jnp.zeros((_L,), jnp.int32) + x


def _build_pack(v, d):
    w = 512
    n_units = v // w // _NW * _NW
    iters = n_units // _NW
    vmain = (v // 128) * 128
    tail_n = v - vmain

    @functools.partial(
        pl.kernel,
        out_type=jax.ShapeDtypeStruct((v // 4, 128), jnp.float32),
        mesh=_MESH,
        compiler_params=pltpu.CompilerParams(use_tc_tiling_on_sc=True, needs_layout_passes=False),
        scratch_types=[
            pltpu.VMEM((d, w), jnp.float32),
            pltpu.VMEM((d, w), jnp.float32),
            pltpu.VMEM((w // 4, 128), jnp.float32),
            pltpu.VMEM((w // 4, 128), jnp.float32),
            pltpu.VMEM((d, tail_n), jnp.float32),
            pltpu.VMEM((tail_n // 4, 128), jnp.float32),
            pltpu.SemaphoreType.DMA,
            pltpu.SemaphoreType.DMA,
            pltpu.SemaphoreType.DMA,
            pltpu.SemaphoreType.DMA,
        ],
    )
    def pack(embt_hbm, tail_hbm, tp_hbm, ib0, ib1, ob0, ob1, ib_t, ob_t,
             gi0, gi1, go0, go1):
        wid = _wid()
        iot = _iota()
        ibs, obs = (ib0, ib1), (ob0, ob1)
        gis, gos = (gi0, gi1), (go0, go1)

        def start_in(k, u):
            return [
                pltpu.async_copy(
                    embt_hbm.at[pl.ds(t * 8, 8), pl.ds(u * w, w)],
                    ibs[k].at[pl.ds(t * 8, 8), :],
                    gis[k],
                )
                for t in range(d // 8)
            ]

        def start_out(k, u):
            return pltpu.async_copy(
                obs[k], tp_hbm.at[pl.ds(u * (w // 4), w // 4), :], gos[k]
            )

        rows01 = (iot, iot + 16)

        def transpose_into(src, dst, n_pr):
            @plsc.parallel_loop(0, n_pr, unroll=8)
            def _(pr):
                cbase = _splat(4 * pr)
                for j in range(8):
                    cols = cbase + (j // 2)
                    dst[pr, pl.ds(j * 16, 16)] = plsc.load_gather(
                        src, [rows01[j % 2], cols]
                    )

        def unit(i):
            return i * _NW + wid

        def drain_in(k):
            for t in range(d // 8):
                pltpu.make_async_copy(
                    embt_hbm.at[pl.ds(t * 8, 8), pl.ds(0, w)],
                    ibs[k].at[pl.ds(t * 8, 8), :],
                    gis[k],
                ).wait()

        def drain_out(k):
            pltpu.make_async_copy(
                obs[k], tp_hbm.at[pl.ds(0, w // 4), :], gos[k]
            ).wait()

        start_in(0, unit(0))
        start_in(1, unit(1))

        def body(i, _):
            for k in range(2):

                @pl.when(i % 2 == k)
                def _():
                    drain_in(k)

                    @pl.when(i >= 2)
                    def _():
                        drain_out(k)

                    transpose_into(ibs[k], obs[k], w // 4)
                    start_out(k, unit(i))

                    @pl.when(i + 2 < iters)
                    def _():
                        start_in(k, unit(i + 2))
            return 0

        lax.fori_loop(0, iters, body, 0)
        drain_out(0)
        drain_out(1)

        @pl.when(wid == _NW - 1)
        def _():
            u = n_units
            for cp in start_in(0, u):
                cp.wait()
            transpose_into(ibs[0], obs[0], w // 4)
            start_out(0, u).wait()
            pltpu.sync_copy(tail_hbm, ib_t)
            transpose_into(ib_t, ob_t, tail_n // 4)
            pltpu.sync_copy(ob_t, tp_hbm.at[pl.ds(vmain // 4, tail_n // 4), :])

    return pack


def _build_gather(n_total, v, d, chunk):
    n_per = n_total // _NW
    n_chunks = n_per // chunk
    assert n_per * _NW == n_total and n_chunks * chunk == n_per

    @functools.partial(
        pl.kernel,
        out_type=jax.ShapeDtypeStruct((n_total, d), jnp.float32),
        mesh=_MESH,
        compiler_params=pltpu.CompilerParams(use_tc_tiling_on_sc=False),
        scratch_types=[
            pltpu.VMEM((n_per,), jnp.int32),
            pltpu.VMEM((chunk, d), jnp.float32),
            pltpu.VMEM((chunk, d), jnp.float32),
            pltpu.SemaphoreType.DMA,
            pltpu.SemaphoreType.DMA,
            pltpu.SemaphoreType.DMA,
            pltpu.SemaphoreType.DMA,
        ],
    )
    def gather(idx_hbm, table_hbm, out_hbm, idx_v, rows0, rows1, g0, g1, o0, o1):
        wid = _wid()
        base = wid * n_per
        pltpu.sync_copy(idx_hbm.at[pl.ds(base, n_per)], idx_v)

        bufs = (rows0, rows1)
        gsems = (g0, g1)
        osems = (o0, o1)

        def start_gather(g):
            b = g % 2
            return pltpu.async_copy(
                table_hbm.at[idx_v.at[pl.ds(g * chunk, chunk)]], bufs[b], gsems[b]
            )

        gcp = start_gather(0)
        prev_out = None
        for g in range(n_chunks):
            b = g % 2
            gcp.wait()
            if g + 1 < n_chunks:
                if prev_out is not None:
                    prev_out.wait()
                gcp = start_gather(g + 1)
            ocp = pltpu.async_copy(
                bufs[b], out_hbm.at[pl.ds(base + g * chunk, chunk)], osems[b]
            )
            if prev_out is not None and g + 1 >= n_chunks:
                prev_out.wait()
            prev_out = ocp
        prev_out.wait()

    return gather


def _build_unpack(b, l, d):
    mb = 4
    n_units = (b * l * d) // (32 * 128)
    per_w = n_units // _NW
    n_blk = per_w // mb
    cb = b // 128

    @functools.partial(
        pl.kernel,
        out_type=jax.ShapeDtypeStruct((l, d, b), jnp.float32),
        mesh=_MESH,
        compiler_params=pltpu.CompilerParams(use_tc_tiling_on_sc=True, needs_layout_passes=False),
        scratch_types=[
            pltpu.VMEM((32 * mb, 128), jnp.float32),
            pltpu.VMEM((32 * mb, 128), jnp.float32),
            pltpu.VMEM((32, 128 * mb), jnp.float32),
            pltpu.VMEM((32, 128 * mb), jnp.float32),
            pltpu.SemaphoreType.DMA,
            pltpu.SemaphoreType.DMA,
            pltpu.SemaphoreType.DMA,
            pltpu.SemaphoreType.DMA,
        ],
    )
    def unpack(rp_hbm, out_hbm, ib0, ib1, ob0, ob1, gi0, gi1, go0, go1):
        wid = _wid()
        iot = _iota()
        ibs, obs = (ib0, ib1), (ob0, ob1)
        gis, gos = (gi0, gi1), (go0, go1)

        def start_in(k, m):
            u0 = wid * per_w + m * mb
            return pltpu.async_copy(
                rp_hbm.at[pl.ds(u0 * 32, 32 * mb), :], ibs[k], gis[k]
            )

        def start_out(k, m):
            u0 = wid * per_w + m * mb
            lu = u0 // cb
            cu = u0 % cb
            return pltpu.async_copy(
                obs[k], out_hbm.at[lu, :, pl.ds(cu * 128, 128 * mb)], gos[k]
            )

        i4 = iot // 4
        colb = (iot % 4) * 32

        def block_transpose(k):
            @plsc.parallel_loop(0, mb * 8, unroll=4)
            def _(q):
                s = q // 8
                g = q % 8
                rows = s * 32 + g * 4 + i4
                off = s * 128 + g * 16
                for t in range(d // 8):
                    for dr in range(8):
                        vals = plsc.load_gather(ibs[k], [rows, colb + (t * 8 + dr)])
                        obs[k][t * 8 + dr, pl.ds(off, 16)] = vals * _SCALE

        ins = [start_in(0, 0), start_in(1, 1)]
        outs = [None, None]

        def body2(i, _):
            for k in range(2):
                m = i * 2 + k
                ins[k].wait()

                @pl.when(m >= 2)
                def _():
                    outs_wait(k)

                block_transpose(k)
                start_out(k, m)

                @pl.when(m + 2 < n_blk)
                def _():
                    pltpu.async_copy(
                        rp_hbm.at[
                            pl.ds((wid * per_w + (m + 2) * mb) * 32, 32 * mb), :
                        ],
                        ibs[k],
                        gis[k],
                    )
            return 0

        def outs_wait(k):
            pltpu.make_async_copy(
                obs[k],
                out_hbm.at[0, :, pl.ds(0, 128 * mb)],
                gos[k],
            ).wait()

        lax.fori_loop(0, n_blk // 2, body2, 0)
        outs_wait(0)
        outs_wait(1)

    return unpack


def kernel(x, emb):
    b, l = x.shape
    v, d = emb.shape
    xf = x.T.reshape(-1).astype(jnp.int32)
    embt = emb.T
    tail = lax.slice(emb, (v - 64, 0), (v, d)).T
    tp = _build_pack(v, d)(embt, tail)
    table = tp.reshape(v, d)
    rows = _build_gather(b * l, v, d, 1024)(xf, table)
    rp = rows.reshape((b * l * d) // 128, 128)
    outt = _build_unpack(b, l, d)(rp)
    return jnp.transpose(outt, (2, 0, 1))


# speedup vs baseline: 1.0449x; 1.0449x over previous
import functools
import math

import jax
import jax.numpy as jnp
from jax import lax
from jax.experimental import pallas as pl
from jax.experimental.pallas import tpu as pltpu
from jax.experimental.pallas import tpu_sc as plsc

_SCALE = float(math.sqrt(32.0))

_NC = 2
_NS = 16
_NW = _NC * _NS
_L = 16

_MESH = plsc.VectorSubcoreMesh(core_axis_name="c", subcore_axis_name="s")


def _wid():
    return lax.axis_index("s") * _NC + lax.axis_index("c")


def _iota():
    return lax.iota(jnp.int32, _L)


def _splat(x):
    return jnp.zeros((_L,), jnp.int32) + x


def _build_pack(v, d):
    w = 512
    n_units = v // w // _NW * _NW
    iters = n_units // _NW
    vmain = (v // 128) * 128
    tail_n = v - vmain

    @functools.partial(
        pl.kernel,
        out_type=jax.ShapeDtypeStruct((v // 4, 128), jnp.float32),
        mesh=_MESH,
        compiler_params=pltpu.CompilerParams(use_tc_tiling_on_sc=True, needs_layout_passes=False),
        scratch_types=[
            pltpu.VMEM((d, w + 1), jnp.float32),
            pltpu.VMEM((d, w + 1), jnp.float32),
            pltpu.VMEM((w // 4, 128), jnp.float32),
            pltpu.VMEM((w // 4, 128), jnp.float32),
            pltpu.VMEM((d, tail_n), jnp.float32),
            pltpu.VMEM((tail_n // 4, 128), jnp.float32),
            pltpu.SemaphoreType.DMA,
            pltpu.SemaphoreType.DMA,
            pltpu.SemaphoreType.DMA,
            pltpu.SemaphoreType.DMA,
        ],
    )
    def pack(embt_hbm, tail_hbm, tp_hbm, ib0, ib1, ob0, ob1, ib_t, ob_t,
             gi0, gi1, go0, go1):
        wid = _wid()
        iot = _iota()
        ibs, obs = (ib0, ib1), (ob0, ob1)
        gis, gos = (gi0, gi1), (go0, go1)

        def start_in(k, u):
            return [
                pltpu.async_copy(
                    embt_hbm.at[pl.ds(t * 8, 8), pl.ds(u * w, w)],
                    ibs[k].at[pl.ds(t * 8, 8), pl.ds(0, w)],
                    gis[k],
                )
                for t in range(d // 8)
            ]

        def start_out(k, u):
            return pltpu.async_copy(
                obs[k], tp_hbm.at[pl.ds(u * (w // 4), w // 4), :], gos[k]
            )

        rows01 = (iot, iot + 16)

        def transpose_into(src, dst, n_pr):
            @plsc.parallel_loop(0, n_pr, unroll=4)
            def _(pr):
                cbase = _splat(4 * pr)
                for j in range(8):
                    cols = cbase + (j // 2)
                    dst[pr, pl.ds(j * 16, 16)] = plsc.load_gather(
                        src, [rows01[j % 2], cols]
                    )

        def unit(i):
            return i * _NW + wid

        def drain_in(k):
            for t in range(d // 8):
                pltpu.make_async_copy(
                    embt_hbm.at[pl.ds(t * 8, 8), pl.ds(0, w)],
                    ibs[k].at[pl.ds(t * 8, 8), pl.ds(0, w)],
                    gis[k],
                ).wait()

        def drain_out(k):
            pltpu.make_async_copy(
                obs[k], tp_hbm.at[pl.ds(0, w // 4), :], gos[k]
            ).wait()

        start_in(0, unit(0))
        start_in(1, unit(1))

        def body(i, _):
            for k in range(2):

                @pl.when(i % 2 == k)
                def _():
                    drain_in(k)

                    @pl.when(i >= 2)
                    def _():
                        drain_out(k)

                    transpose_into(ibs[k], obs[k], w // 4)
                    start_out(k, unit(i))

                    @pl.when(i + 2 < iters)
                    def _():
                        start_in(k, unit(i + 2))
            return 0

        lax.fori_loop(0, iters, body, 0)
        drain_out(0)
        drain_out(1)

        @pl.when(wid == _NW - 1)
        def _():
            u = n_units
            for cp in start_in(0, u):
                cp.wait()
            transpose_into(ibs[0], obs[0], w // 4)
            start_out(0, u).wait()
            pltpu.sync_copy(tail_hbm, ib_t)
            transpose_into(ib_t, ob_t, tail_n // 4)
            pltpu.sync_copy(ob_t, tp_hbm.at[pl.ds(vmain // 4, tail_n // 4), :])

    return pack


def _build_gather(n_total, v, d, chunk):
    n_per = n_total // _NW
    n_chunks = n_per // chunk
    assert n_per * _NW == n_total and n_chunks * chunk == n_per

    @functools.partial(
        pl.kernel,
        out_type=jax.ShapeDtypeStruct((n_total, d), jnp.float32),
        mesh=_MESH,
        compiler_params=pltpu.CompilerParams(use_tc_tiling_on_sc=False),
        scratch_types=[
            pltpu.VMEM((n_per,), jnp.int32),
            pltpu.VMEM((chunk, d), jnp.float32),
            pltpu.VMEM((chunk, d), jnp.float32),
            pltpu.SemaphoreType.DMA,
            pltpu.SemaphoreType.DMA,
            pltpu.SemaphoreType.DMA,
            pltpu.SemaphoreType.DMA,
        ],
    )
    def gather(idx_hbm, table_hbm, out_hbm, idx_v, rows0, rows1, g0, g1, o0, o1):
        wid = _wid()
        base = wid * n_per
        pltpu.sync_copy(idx_hbm.at[pl.ds(base, n_per)], idx_v)

        bufs = (rows0, rows1)
        gsems = (g0, g1)
        osems = (o0, o1)

        def start_gather(g):
            b = g % 2
            return pltpu.async_copy(
                table_hbm.at[idx_v.at[pl.ds(g * chunk, chunk)]], bufs[b], gsems[b]
            )

        gcp = start_gather(0)
        prev_out = None
        for g in range(n_chunks):
            b = g % 2
            gcp.wait()
            if g + 1 < n_chunks:
                if prev_out is not None:
                    prev_out.wait()
                gcp = start_gather(g + 1)
            ocp = pltpu.async_copy(
                bufs[b], out_hbm.at[pl.ds(base + g * chunk, chunk)], osems[b]
            )
            if prev_out is not None and g + 1 >= n_chunks:
                prev_out.wait()
            prev_out = ocp
        prev_out.wait()

    return gather


def _build_unpack(b, l, d):
    mb = 4
    n_units = (b * l * d) // (32 * 128)
    per_w = n_units // _NW
    n_blk = per_w // mb
    cb = b // 128

    @functools.partial(
        pl.kernel,
        out_type=jax.ShapeDtypeStruct((l, d, b), jnp.float32),
        mesh=_MESH,
        compiler_params=pltpu.CompilerParams(use_tc_tiling_on_sc=True, needs_layout_passes=False),
        scratch_types=[
            pltpu.VMEM((32 * mb, 128), jnp.float32),
            pltpu.VMEM((32 * mb, 128), jnp.float32),
            pltpu.VMEM((32, 128 * mb + 1), jnp.float32),
            pltpu.VMEM((32, 128 * mb + 1), jnp.float32),
            pltpu.SemaphoreType.DMA,
            pltpu.SemaphoreType.DMA,
            pltpu.SemaphoreType.DMA,
            pltpu.SemaphoreType.DMA,
        ],
    )
    def unpack(rp_hbm, out_hbm, ib0, ib1, ob0, ob1, gi0, gi1, go0, go1):
        wid = _wid()
        iot = _iota()
        ibs, obs = (ib0, ib1), (ob0, ob1)
        gis, gos = (gi0, gi1), (go0, go1)

        def start_in(k, m):
            u0 = wid * per_w + m * mb
            return pltpu.async_copy(
                rp_hbm.at[pl.ds(u0 * 32, 32 * mb), :], ibs[k], gis[k]
            )

        def start_out(k, m):
            u0 = wid * per_w + m * mb
            lu = u0 // cb
            cu = u0 % cb
            return pltpu.async_copy(
                obs[k].at[:, pl.ds(0, 128 * mb)],
                out_hbm.at[lu, :, pl.ds(cu * 128, 128 * mb)],
                gos[k],
            )

        rows01 = (iot, iot + 16)

        def block_transpose(k):
            @plsc.parallel_loop(0, 32 * mb, unroll=2)
            def _(pr):
                s = pr // 32
                rr = pr % 32
                cb0 = s * 128 + 4 * rr
                for seg in range(8):
                    vals = ibs[k][pr, pl.ds(seg * 16, 16)] * _SCALE
                    cols = _splat(cb0 + (seg // 2))
                    plsc.store_scatter(obs[k], [rows01[seg % 2], cols], vals)

        ins = [start_in(0, 0), start_in(1, 1)]
        outs = [None, None]

        def body2(i, _):
            for k in range(2):
                m = i * 2 + k
                ins[k].wait()

                @pl.when(m >= 2)
                def _():
                    outs_wait(k)

                block_transpose(k)
                start_out(k, m)

                @pl.when(m + 2 < n_blk)
                def _():
                    pltpu.async_copy(
                        rp_hbm.at[
                            pl.ds((wid * per_w + (m + 2) * mb) * 32, 32 * mb), :
                        ],
                        ibs[k],
                        gis[k],
                    )
            return 0

        def outs_wait(k):
            pltpu.make_async_copy(
                obs[k].at[:, pl.ds(0, 128 * mb)],
                out_hbm.at[0, :, pl.ds(0, 128 * mb)],
                gos[k],
            ).wait()

        lax.fori_loop(0, n_blk // 2, body2, 0)
        outs_wait(0)
        outs_wait(1)

    return unpack


def kernel(x, emb):
    b, l = x.shape
    v, d = emb.shape
    xf = x.T.reshape(-1).astype(jnp.int32)
    embt = emb.T
    tail = lax.slice(emb, (v - 64, 0), (v, d)).T
    tp = _build_pack(v, d)(embt, tail)
    table = tp.reshape(v, d)
    rows = _build_gather(b * l, v, d, 1024)(xf, table)
    rp = rows.reshape((b * l * d) // 128, 128)
    outt = _build_unpack(b, l, d)(rp)
    return jnp.transpose(outt, (2, 0, 1))


# speedup vs baseline: 1.0666x; 1.0208x over previous
import functools
import math

import jax
import jax.numpy as jnp
from jax import lax
from jax.experimental import pallas as pl
from jax.experimental.pallas import tpu as pltpu
from jax.experimental.pallas import tpu_sc as plsc

_SCALE = float(math.sqrt(32.0))

_NC = 2
_NS = 16
_NW = _NC * _NS
_L = 16

_MESH = plsc.VectorSubcoreMesh(core_axis_name="c", subcore_axis_name="s")


def _wid():
    return lax.axis_index("s") * _NC + lax.axis_index("c")


def _iota():
    return lax.iota(jnp.int32, _L)


def _splat(x):
    return jnp.zeros((_L,), jnp.int32) + x


def _build_pack(v, d):
    w = 512
    n_units = v // w // _NW * _NW
    iters = n_units // _NW
    vmain = (v // 128) * 128
    tail_n = v - vmain

    @functools.partial(
        pl.kernel,
        out_type=jax.ShapeDtypeStruct((v // 4, 128), jnp.float32),
        mesh=_MESH,
        compiler_params=pltpu.CompilerParams(use_tc_tiling_on_sc=True, needs_layout_passes=False),
        scratch_types=[
            pltpu.VMEM((d, w + 1), jnp.float32),
            pltpu.VMEM((d, w + 1), jnp.float32),
            pltpu.VMEM((w // 4, 128), jnp.float32),
            pltpu.VMEM((w // 4, 128), jnp.float32),
            pltpu.VMEM((d, tail_n), jnp.float32),
            pltpu.VMEM((tail_n // 4, 128), jnp.float32),
            pltpu.SemaphoreType.DMA,
            pltpu.SemaphoreType.DMA,
            pltpu.SemaphoreType.DMA,
            pltpu.SemaphoreType.DMA,
        ],
    )
    def pack(embt_hbm, tail_hbm, tp_hbm, ib0, ib1, ob0, ob1, ib_t, ob_t,
             gi0, gi1, go0, go1):
        wid = _wid()
        iot = _iota()
        ibs, obs = (ib0, ib1), (ob0, ob1)
        gis, gos = (gi0, gi1), (go0, go1)

        def start_in(k, u):
            return [
                pltpu.async_copy(
                    embt_hbm.at[pl.ds(t * 8, 8), pl.ds(u * w, w)],
                    ibs[k].at[pl.ds(t * 8, 8), pl.ds(0, w)],
                    gis[k],
                )
                for t in range(d // 8)
            ]

        def start_out(k, u):
            return pltpu.async_copy(
                obs[k], tp_hbm.at[pl.ds(u * (w // 4), w // 4), :], gos[k]
            )

        rows01 = (iot, iot + 16)

        def transpose_into(src, dst, n_pr):
            @plsc.parallel_loop(0, n_pr, unroll=4)
            def _(pr):
                cbase = _splat(4 * pr)
                for j in range(8):
                    cols = cbase + (j // 2)
                    dst[pr, pl.ds(j * 16, 16)] = plsc.load_gather(
                        src, [rows01[j % 2], cols]
                    )

        def unit(i):
            return i * _NW + wid

        def drain_in(k):
            for t in range(d // 8):
                pltpu.make_async_copy(
                    embt_hbm.at[pl.ds(t * 8, 8), pl.ds(0, w)],
                    ibs[k].at[pl.ds(t * 8, 8), pl.ds(0, w)],
                    gis[k],
                ).wait()

        def drain_out(k):
            pltpu.make_async_copy(
                obs[k], tp_hbm.at[pl.ds(0, w // 4), :], gos[k]
            ).wait()

        start_in(0, unit(0))
        start_in(1, unit(1))

        def body(i, _):
            for k in range(2):

                @pl.when(i % 2 == k)
                def _():
                    drain_in(k)

                    @pl.when(i >= 2)
                    def _():
                        drain_out(k)

                    transpose_into(ibs[k], obs[k], w // 4)
                    start_out(k, unit(i))

                    @pl.when(i + 2 < iters)
                    def _():
                        start_in(k, unit(i + 2))
            return 0

        lax.fori_loop(0, iters, body, 0)
        drain_out(0)
        drain_out(1)

        @pl.when(wid == _NW - 1)
        def _():
            u = n_units
            for cp in start_in(0, u):
                cp.wait()
            transpose_into(ibs[0], obs[0], w // 4)
            start_out(0, u).wait()
            pltpu.sync_copy(tail_hbm, ib_t)
            transpose_into(ib_t, ob_t, tail_n // 4)
            pltpu.sync_copy(ob_t, tp_hbm.at[pl.ds(vmain // 4, tail_n // 4), :])

    return pack


def _build_gather(n_total, v, d, chunk):
    n_per = n_total // _NW
    n_chunks = n_per // chunk
    assert n_per * _NW == n_total and n_chunks * chunk == n_per

    @functools.partial(
        pl.kernel,
        out_type=jax.ShapeDtypeStruct((n_total, d), jnp.float32),
        mesh=_MESH,
        compiler_params=pltpu.CompilerParams(use_tc_tiling_on_sc=False),
        scratch_types=[
            pltpu.VMEM((n_per,), jnp.int32),
            pltpu.VMEM((chunk, d), jnp.float32),
            pltpu.VMEM((chunk, d), jnp.float32),
            pltpu.SemaphoreType.DMA,
            pltpu.SemaphoreType.DMA,
            pltpu.SemaphoreType.DMA,
            pltpu.SemaphoreType.DMA,
        ],
    )
    def gather(idx_hbm, table_hbm, out_hbm, idx_v, rows0, rows1, g0, g1, o0, o1):
        wid = _wid()
        base = wid * n_per
        pltpu.sync_copy(idx_hbm.at[pl.ds(base, n_per)], idx_v)

        bufs = (rows0, rows1)
        gsems = (g0, g1)
        osems = (o0, o1)

        def start_gather(g):
            b = g % 2
            return pltpu.async_copy(
                table_hbm.at[idx_v.at[pl.ds(g * chunk, chunk)]], bufs[b], gsems[b]
            )

        gcp = start_gather(0)
        prev_out = None
        for g in range(n_chunks):
            b = g % 2
            gcp.wait()
            if g + 1 < n_chunks:
                if prev_out is not None:
                    prev_out.wait()
                gcp = start_gather(g + 1)
            ocp = pltpu.async_copy(
                bufs[b], out_hbm.at[pl.ds(base + g * chunk, chunk)], osems[b]
            )
            if prev_out is not None and g + 1 >= n_chunks:
                prev_out.wait()
            prev_out = ocp
        prev_out.wait()

    return gather


def _build_unpack(b, l, d):
    mb = 4
    n_units = (b * l * d) // (32 * 128)
    per_w = n_units // _NW
    n_blk = per_w // mb
    cb = b // 128

    @functools.partial(
        pl.kernel,
        out_type=jax.ShapeDtypeStruct((l, d, b), jnp.float32),
        mesh=_MESH,
        compiler_params=pltpu.CompilerParams(use_tc_tiling_on_sc=True, needs_layout_passes=False),
        scratch_types=[
            pltpu.VMEM((32 * mb, 128), jnp.float32),
            pltpu.VMEM((32 * mb, 128), jnp.float32),
            pltpu.VMEM((32, 128 * mb), jnp.float32),
            pltpu.VMEM((32, 128 * mb), jnp.float32),
            pltpu.SemaphoreType.DMA,
            pltpu.SemaphoreType.DMA,
            pltpu.SemaphoreType.DMA,
            pltpu.SemaphoreType.DMA,
        ],
    )
    def unpack(rp_hbm, out_hbm, ib0, ib1, ob0, ob1, gi0, gi1, go0, go1):
        wid = _wid()
        iot = _iota()
        ibs, obs = (ib0, ib1), (ob0, ob1)
        gis, gos = (gi0, gi1), (go0, go1)

        def start_in(k, m):
            u0 = wid * per_w + m * mb
            return pltpu.async_copy(
                rp_hbm.at[pl.ds(u0 * 32, 32 * mb), :], ibs[k], gis[k]
            )

        def start_out(k, m):
            u0 = wid * per_w + m * mb
            lu = u0 // cb
            cu = u0 % cb
            return pltpu.async_copy(
                obs[k], out_hbm.at[lu, :, pl.ds(cu * 128, 128 * mb)], gos[k]
            )

        i4 = iot // 4
        colb = (iot % 4) * 32

        def block_transpose(k):
            @plsc.parallel_loop(0, mb * 8, unroll=2)
            def _(q):
                s = q // 8
                g = q % 8
                rows = s * 32 + g * 4 + i4
                off = s * 128 + g * 16
                for t in range(d // 8):
                    for dr in range(8):
                        vals = plsc.load_gather(ibs[k], [rows, colb + (t * 8 + dr)])
                        obs[k][t * 8 + dr, pl.ds(off, 16)] = vals * _SCALE

        ins = [start_in(0, 0), start_in(1, 1)]
        outs = [None, None]

        def body2(i, _):
            for k in range(2):
                m = i * 2 + k
                ins[k].wait()

                @pl.when(m >= 2)
                def _():
                    outs_wait(k)

                block_transpose(k)
                start_out(k, m)

                @pl.when(m + 2 < n_blk)
                def _():
                    pltpu.async_copy(
                        rp_hbm.at[
                            pl.ds((wid * per_w + (m + 2) * mb) * 32, 32 * mb), :
                        ],
                        ibs[k],
                        gis[k],
                    )
            return 0

        def outs_wait(k):
            pltpu.make_async_copy(
                obs[k],
                out_hbm.at[0, :, pl.ds(0, 128 * mb)],
                gos[k],
            ).wait()

        lax.fori_loop(0, n_blk // 2, body2, 0)
        outs_wait(0)
        outs_wait(1)

    return unpack


def kernel(x, emb):
    b, l = x.shape
    v, d = emb.shape
    xf = x.T.reshape(-1).astype(jnp.int32)
    embt = emb.T
    tail = lax.slice(emb, (v - 64, 0), (v, d)).T
    tp = _build_pack(v, d)(embt, tail)
    table = tp.reshape(v, d)
    rows = _build_gather(b * l, v, d, 1024)(xf, table)
    rp = rows.reshape((b * l * d) // 128, 128)
    outt = _build_unpack(b, l, d)(rp)
    return jnp.transpose(outt, (2, 0, 1))
